# Initial kernel scaffold; baseline (speedup 1.0000x reference)
import functools

import jax
import jax.numpy as jnp
from jax import lax
from jax.experimental import pallas as pl
from jax.experimental.pallas import tpu as pltpu
from jax.experimental.pallas import tpu_sc as plsc

N = 10000
E = 320000
D_IN = 128
LANES = 16
NC, NS = 2, 16
NW = NC * NS
CHUNK = 128
NCHUNK = 79
EW = NCHUNK * CHUNK
EPAD = NW * EW
NPAD = 10016
RPS = NPAD // NS

_mesh = plsc.VectorSubcoreMesh(core_axis_name="c", subcore_axis_name="s")
_f32 = jnp.float32
_node_rows = jax.ShapeDtypeStruct((NPAD, LANES), _f32)


@functools.partial(
    pl.kernel,
    out_type=(_node_rows,) * 4,
    mesh=_mesh,
    scratch_types=[
        pltpu.VMEM((NCHUNK, CHUNK), jnp.int32),
        pltpu.VMEM((NCHUNK, CHUNK), jnp.int32),
        pltpu.VMEM((CHUNK, LANES), _f32),
        pltpu.VMEM((RPS, LANES), _f32),
        pltpu.VMEM_SHARED((NPAD, LANES), _f32),
        pltpu.VMEM_SHARED((NPAD, LANES), _f32),
    ],
)
def _deg_kernel(a_hbm, b_hbm, dega0, dega1, degb0, degb1,
                a_v, b_v, ones_v, stage_v, acc_a, acc_b):
    c = lax.axis_index("c")
    s = lax.axis_index("s")
    w = c * NS + s
    row0 = s * RPS

    def fill_ones(i, carry):
        ones_v[i, :] = jnp.ones((LANES,), _f32)
        return carry

    lax.fori_loop(0, CHUNK, fill_ones, 0)

    def fill_zero(i, carry):
        stage_v[i, :] = jnp.zeros((LANES,), _f32)
        return carry

    lax.fori_loop(0, RPS, fill_zero, 0)
    pltpu.sync_copy(stage_v, acc_a.at[pl.ds(row0, RPS)])
    pltpu.sync_copy(stage_v, acc_b.at[pl.ds(row0, RPS)])
    pltpu.sync_copy(a_hbm.at[w], a_v)
    pltpu.sync_copy(b_hbm.at[w], b_v)
    plsc.subcore_barrier()

    def chunk_body(i, carry):
        pltpu.sync_copy(ones_v, acc_a.at[a_v.at[i]], add=True)
        pltpu.sync_copy(ones_v, acc_b.at[b_v.at[i]], add=True)
        return carry

    lax.fori_loop(0, NCHUNK, chunk_body, 0)
    plsc.subcore_barrier()

    @pl.when(c == 0)
    def _():
        pltpu.sync_copy(acc_a.at[pl.ds(row0, RPS)], stage_v)
        pltpu.sync_copy(stage_v, dega0.at[pl.ds(row0, RPS)])
        pltpu.sync_copy(acc_b.at[pl.ds(row0, RPS)], stage_v)
        pltpu.sync_copy(stage_v, degb0.at[pl.ds(row0, RPS)])

    @pl.when(c == 1)
    def _():
        pltpu.sync_copy(acc_a.at[pl.ds(row0, RPS)], stage_v)
        pltpu.sync_copy(stage_v, dega1.at[pl.ds(row0, RPS)])
        pltpu.sync_copy(acc_b.at[pl.ds(row0, RPS)], stage_v)
        pltpu.sync_copy(stage_v, degb1.at[pl.ds(row0, RPS)])


@functools.partial(
    pl.kernel,
    out_type=(_node_rows,) * 2,
    mesh=_mesh,
    scratch_types=[
        pltpu.VMEM((NCHUNK, CHUNK), jnp.int32),
        pltpu.VMEM((NCHUNK, CHUNK), jnp.int32),
        pltpu.VMEM((CHUNK, LANES), _f32),
        pltpu.VMEM((RPS, LANES), _f32),
        pltpu.VMEM_SHARED((NPAD, LANES), _f32),
        pltpu.VMEM_SHARED((NPAD, LANES), _f32),
    ],
)
def _edge_scatter(g_hbm, src_hbm, dst_hbm, out0, out1,
                  src_v, dst_v, rows_v, stage_v, g_sm, acc_sm):
    c = lax.axis_index("c")
    s = lax.axis_index("s")
    w = c * NS + s
    row0 = s * RPS

    pltpu.sync_copy(g_hbm.at[pl.ds(row0, RPS)], stage_v)
    pltpu.sync_copy(stage_v, g_sm.at[pl.ds(row0, RPS)])
    pltpu.sync_copy(stage_v, acc_sm.at[pl.ds(row0, RPS)])
    pltpu.sync_copy(src_hbm.at[w], src_v)
    pltpu.sync_copy(dst_hbm.at[w], dst_v)
    plsc.subcore_barrier()

    def chunk_body(i, carry):
        pltpu.sync_copy(g_sm.at[src_v.at[i]], rows_v)
        pltpu.sync_copy(rows_v, acc_sm.at[dst_v.at[i]], add=True)
        return carry

    lax.fori_loop(0, NCHUNK, chunk_body, 0)
    plsc.subcore_barrier()

    @pl.when(c == 0)
    def _():
        pltpu.sync_copy(acc_sm.at[pl.ds(row0, RPS)], stage_v)
        pltpu.sync_copy(stage_v, out0.at[pl.ds(row0, RPS)])

    @pl.when(c == 1)
    def _():
        pltpu.sync_copy(acc_sm.at[pl.ds(row0, RPS)], stage_v)
        pltpu.sync_copy(stage_v, out1.at[pl.ds(row0, RPS)])


def _k1_body(x_ref, w1_ref, da0, da1, db0, db1,
             g1_ref, dinv_rev_ref, dinv_fwd_ref):
    rows = lax.broadcasted_iota(jnp.int32, (NPAD, LANES), 0)
    valid = jnp.where(rows < N, 1.0, 0.0).astype(_f32)
    dinv_rev = valid * lax.rsqrt(da0[...] + da1[...] + 1.0)
    dinv_fwd = valid * lax.rsqrt(db0[...] + db1[...] + 1.0)
    z1 = jnp.dot(x_ref[...], w1_ref[...], preferred_element_type=_f32)
    g1_ref[...] = dinv_rev * z1
    dinv_rev_ref[...] = dinv_rev
    dinv_fwd_ref[...] = dinv_fwd


def _kmid_body(p0, p1, gp, dprev, dnext, b_ref, w_ref, gn_ref):
    p = dprev[...] * (p0[...] + p1[...] - gp[...])
    act = jnp.maximum(p + b_ref[...], 0.0)
    gn_ref[...] = dnext[...] * jnp.dot(act, w_ref[...],
                                       preferred_element_type=_f32)


def _k5_body(p0, p1, g4, dinv, b_ref, wo_ref, wv_ref, wd_ref,
             bv_ref, bd_ref, g5_ref, v_ref, pn_ref):
    p = dinv[...] * (p0[...] + p1[...] - g4[...])
    hfin = jnp.maximum(p + b_ref[...], 0.0)
    g5_ref[...] = dinv[...] * jnp.dot(hfin, wo_ref[...],
                                      preferred_element_type=_f32)
    rows = lax.broadcasted_iota(jnp.int32, (NPAD, LANES), 0)
    hmask = jnp.where(rows < N, hfin, 0.0)
    xmean = jnp.sum(hmask, axis=0, keepdims=True) * (1.0 / N)
    v_ref[...] = jnp.dot(xmean, wv_ref[...],
                         preferred_element_type=_f32) + bv_ref[...]
    pn_ref[...] = jnp.dot(xmean, wd_ref[...],
                          preferred_element_type=_f32) + bd_ref[...]


def _k6_body(p0, p1, g5, dinv, bo_ref, out_ref):
    out_ref[...] = dinv[...] * (p0[...] + p1[...] - g5[...]) + bo_ref[...]


def _pad_w(w):
    out = jnp.zeros((LANES, LANES), _f32)
    return out.at[: w.shape[0], : w.shape[1]].set(w)


def _pad_b(b):
    out = jnp.zeros((1, LANES), _f32)
    return out.at[0, : b.shape[0]].set(b)


def kernel(x, edge_index, W1, b1, W2, b2, Wp, bp, W3, b3, Wo, bo,
           Wd, bd, Wv, bv):
    ei_a = edge_index[0]
    ei_b = edge_index[1]
    pad = N + (jnp.arange(EPAD - E, dtype=jnp.int32) % LANES)
    a3 = jnp.concatenate([ei_a, pad]).reshape(NW, NCHUNK, CHUNK)
    b3 = jnp.concatenate([ei_b, pad]).reshape(NW, NCHUNK, CHUNK)
    xp = jnp.zeros((NPAD, D_IN), _f32).at[:N].set(x)
    w1p = jnp.zeros((D_IN, LANES), _f32).at[:, :10].set(W1)
    w2p, wpp, w3p = _pad_w(W2), _pad_w(Wp), _pad_w(W3)
    wop, wvp, wdp = _pad_w(Wo), _pad_w(Wv), _pad_w(Wd)
    b1p, b2p, bpp, b3p = _pad_b(b1), _pad_b(b2), _pad_b(bp), _pad_b(b3)
    bop, bvp, bdp = _pad_b(bo), _pad_b(bv), _pad_b(bd)

    da0, da1, db0, db1 = _deg_kernel(a3, b3)
    g1, dinv_rev, dinv_fwd = pl.pallas_call(
        _k1_body, out_shape=(_node_rows,) * 3,
    )(xp, w1p, da0, da1, db0, db1)

    kmid = pl.pallas_call(_kmid_body, out_shape=_node_rows)

    s0, s1 = _edge_scatter(g1, b3, a3)
    g2 = kmid(s0, s1, g1, dinv_rev, dinv_rev, b1p, w2p)
    s0, s1 = _edge_scatter(g2, b3, a3)
    g3 = kmid(s0, s1, g2, dinv_rev, dinv_fwd, b2p, wpp)
    s0, s1 = _edge_scatter(g3, a3, b3)
    g4 = kmid(s0, s1, g3, dinv_fwd, dinv_rev, bpp, w3p)
    s0, s1 = _edge_scatter(g4, b3, a3)
    g5, vhead, pnhead = pl.pallas_call(
        _k5_body,
        out_shape=(_node_rows,
                   jax.ShapeDtypeStruct((1, LANES), _f32),
                   jax.ShapeDtypeStruct((1, LANES), _f32)),
    )(s0, s1, g4, dinv_rev, b3p, wop, wvp, wdp, bvp, bdp)
    s0, s1 = _edge_scatter(g5, b3, a3)
    p5 = pl.pallas_call(_k6_body, out_shape=_node_rows)(
        s0, s1, g5, dinv_rev, bop)

    probs = jnp.concatenate([p5[:N, :1], pnhead[:1, :1]], axis=0)
    return (probs, vhead[0, :1])


# speedup vs baseline: 51.5373x; 51.5373x over previous
import functools

import jax
import jax.numpy as jnp
from jax import lax
from jax.experimental import pallas as pl
from jax.experimental.pallas import tpu as pltpu
from jax.experimental.pallas import tpu_sc as plsc

N = 10000
E = 320000
D_IN = 128
LANES = 16
NC, NS = 2, 16
NW = NC * NS
CHUNK = 128
NCHUNK = 79
EW = NCHUNK * CHUNK
EPAD = NW * EW
NPAD = 10112
RPS = NPAD // NS

_mesh = plsc.VectorSubcoreMesh(core_axis_name="c", subcore_axis_name="s")
_f32 = jnp.float32
_node_rows = jax.ShapeDtypeStruct((NPAD, LANES), _f32)


@functools.partial(
    pl.kernel,
    out_type=(_node_rows,) * 4,
    mesh=_mesh,
    compiler_params=pltpu.CompilerParams(use_tc_tiling_on_sc=False),
    scratch_types=[
        pltpu.VMEM((NCHUNK, CHUNK), jnp.int32),
        pltpu.VMEM((NCHUNK, CHUNK), jnp.int32),
        pltpu.VMEM((CHUNK, LANES), _f32),
        pltpu.VMEM((RPS, LANES), _f32),
        pltpu.VMEM_SHARED((NPAD, LANES), _f32),
        pltpu.VMEM_SHARED((NPAD, LANES), _f32),
    ],
)
def _deg_kernel(a_hbm, b_hbm, dega0, dega1, degb0, degb1,
                a_v, b_v, ones_v, stage_v, acc_a, acc_b):
    c = lax.axis_index("c")
    s = lax.axis_index("s")
    w = c * NS + s
    row0 = pl.multiple_of(s * RPS, 8)

    def fill_ones(i, carry):
        ones_v[i, :] = jnp.ones((LANES,), _f32)
        return carry

    lax.fori_loop(0, CHUNK, fill_ones, 0)

    def fill_zero(i, carry):
        stage_v[i, :] = jnp.zeros((LANES,), _f32)
        return carry

    lax.fori_loop(0, RPS, fill_zero, 0)
    pltpu.sync_copy(stage_v, acc_a.at[pl.ds(row0, RPS)])
    pltpu.sync_copy(stage_v, acc_b.at[pl.ds(row0, RPS)])
    pltpu.sync_copy(a_hbm.at[w], a_v)
    pltpu.sync_copy(b_hbm.at[w], b_v)
    plsc.subcore_barrier()

    def chunk_body(i, carry):
        pltpu.sync_copy(ones_v, acc_a.at[a_v.at[i]], add=True)
        pltpu.sync_copy(ones_v, acc_b.at[b_v.at[i]], add=True)
        return carry

    lax.fori_loop(0, NCHUNK, chunk_body, 0)
    plsc.subcore_barrier()

    @pl.when(c == 0)
    def _():
        pltpu.sync_copy(acc_a.at[pl.ds(row0, RPS)], stage_v)
        pltpu.sync_copy(stage_v, dega0.at[pl.ds(row0, RPS)])
        pltpu.sync_copy(acc_b.at[pl.ds(row0, RPS)], stage_v)
        pltpu.sync_copy(stage_v, degb0.at[pl.ds(row0, RPS)])

    @pl.when(c == 1)
    def _():
        pltpu.sync_copy(acc_a.at[pl.ds(row0, RPS)], stage_v)
        pltpu.sync_copy(stage_v, dega1.at[pl.ds(row0, RPS)])
        pltpu.sync_copy(acc_b.at[pl.ds(row0, RPS)], stage_v)
        pltpu.sync_copy(stage_v, degb1.at[pl.ds(row0, RPS)])


@functools.partial(
    pl.kernel,
    out_type=(_node_rows,) * 2,
    mesh=_mesh,
    compiler_params=pltpu.CompilerParams(use_tc_tiling_on_sc=False),
    scratch_types=[
        pltpu.VMEM((NCHUNK, CHUNK), jnp.int32),
        pltpu.VMEM((NCHUNK, CHUNK), jnp.int32),
        pltpu.VMEM((CHUNK, LANES), _f32),
        pltpu.VMEM((RPS, LANES), _f32),
        pltpu.VMEM_SHARED((NPAD, LANES), _f32),
        pltpu.VMEM_SHARED((NPAD, LANES), _f32),
    ],
)
def _edge_scatter(g_hbm, src_hbm, dst_hbm, out0, out1,
                  src_v, dst_v, rows_v, stage_v, g_sm, acc_sm):
    c = lax.axis_index("c")
    s = lax.axis_index("s")
    w = c * NS + s
    row0 = pl.multiple_of(s * RPS, 8)

    pltpu.sync_copy(g_hbm.at[pl.ds(row0, RPS)], stage_v)
    pltpu.sync_copy(stage_v, g_sm.at[pl.ds(row0, RPS)])
    pltpu.sync_copy(stage_v, acc_sm.at[pl.ds(row0, RPS)])
    pltpu.sync_copy(src_hbm.at[w], src_v)
    pltpu.sync_copy(dst_hbm.at[w], dst_v)
    plsc.subcore_barrier()

    def chunk_body(i, carry):
        pltpu.sync_copy(g_sm.at[src_v.at[i]], rows_v)
        pltpu.sync_copy(rows_v, acc_sm.at[dst_v.at[i]], add=True)
        return carry

    lax.fori_loop(0, NCHUNK, chunk_body, 0)
    plsc.subcore_barrier()

    @pl.when(c == 0)
    def _():
        pltpu.sync_copy(acc_sm.at[pl.ds(row0, RPS)], stage_v)
        pltpu.sync_copy(stage_v, out0.at[pl.ds(row0, RPS)])

    @pl.when(c == 1)
    def _():
        pltpu.sync_copy(acc_sm.at[pl.ds(row0, RPS)], stage_v)
        pltpu.sync_copy(stage_v, out1.at[pl.ds(row0, RPS)])


def _k1_body(x_ref, w1_ref, da0, da1, db0, db1,
             g1_ref, dinv_rev_ref, dinv_fwd_ref):
    rows = lax.broadcasted_iota(jnp.int32, (NPAD, LANES), 0)
    valid = jnp.where(rows < N, 1.0, 0.0).astype(_f32)
    dinv_rev = valid * lax.rsqrt(da0[...] + da1[...] + 1.0)
    dinv_fwd = valid * lax.rsqrt(db0[...] + db1[...] + 1.0)
    z1 = jnp.dot(x_ref[...], w1_ref[...], preferred_element_type=_f32)
    g1_ref[...] = dinv_rev * z1
    dinv_rev_ref[...] = dinv_rev
    dinv_fwd_ref[...] = dinv_fwd


def _kmid_body(p0, p1, gp, dprev, dnext, b_ref, w_ref, gn_ref):
    p = dprev[...] * (p0[...] + p1[...] - gp[...])
    act = jnp.maximum(p + b_ref[...], 0.0)
    gn_ref[...] = dnext[...] * jnp.dot(act, w_ref[...],
                                       preferred_element_type=_f32)


def _k5_body(p0, p1, g4, dinv, b_ref, wo_ref, wv_ref, wd_ref,
             bv_ref, bd_ref, g5_ref, v_ref, pn_ref):
    p = dinv[...] * (p0[...] + p1[...] - g4[...])
    hfin = jnp.maximum(p + b_ref[...], 0.0)
    g5_ref[...] = dinv[...] * jnp.dot(hfin, wo_ref[...],
                                      preferred_element_type=_f32)
    rows = lax.broadcasted_iota(jnp.int32, (NPAD, LANES), 0)
    hmask = jnp.where(rows < N, hfin, 0.0)
    xmean = jnp.sum(hmask, axis=0, keepdims=True) * (1.0 / N)
    v_ref[...] = jnp.dot(xmean, wv_ref[...],
                         preferred_element_type=_f32) + bv_ref[...]
    pn_ref[...] = jnp.dot(xmean, wd_ref[...],
                          preferred_element_type=_f32) + bd_ref[...]


def _k6_body(p0, p1, g5, dinv, bo_ref, out_ref):
    out_ref[...] = dinv[...] * (p0[...] + p1[...] - g5[...]) + bo_ref[...]


def _pad_w(w):
    out = jnp.zeros((LANES, LANES), _f32)
    return out.at[: w.shape[0], : w.shape[1]].set(w)


def _pad_b(b):
    out = jnp.zeros((1, LANES), _f32)
    return out.at[0, : b.shape[0]].set(b)


def kernel(x, edge_index, W1, b1, W2, b2, Wp, bp, W3, b3, Wo, bo,
           Wd, bd, Wv, bv):
    ei_a = edge_index[0]
    ei_b = edge_index[1]
    pad = N + (jnp.arange(EPAD - E, dtype=jnp.int32) % LANES)
    ea3 = jnp.concatenate([ei_a, pad]).reshape(NW, NCHUNK, CHUNK)
    eb3 = jnp.concatenate([ei_b, pad]).reshape(NW, NCHUNK, CHUNK)
    xp = jnp.zeros((NPAD, D_IN), _f32).at[:N].set(x)
    w1p = jnp.zeros((D_IN, LANES), _f32).at[:, :10].set(W1)
    w2p, wpp, w3p = _pad_w(W2), _pad_w(Wp), _pad_w(W3)
    wop, wvp, wdp = _pad_w(Wo), _pad_w(Wv), _pad_w(Wd)
    b1p, b2p, bpp, b3p = _pad_b(b1), _pad_b(b2), _pad_b(bp), _pad_b(b3)
    bop, bvp, bdp = _pad_b(bo), _pad_b(bv), _pad_b(bd)

    da0, da1, db0, db1 = _deg_kernel(ea3, eb3)
    g1, dinv_rev, dinv_fwd = pl.pallas_call(
        _k1_body, out_shape=(_node_rows,) * 3,
    )(xp, w1p, da0, da1, db0, db1)

    kmid = pl.pallas_call(_kmid_body, out_shape=_node_rows)

    s0, s1 = _edge_scatter(g1, eb3, ea3)
    g2 = kmid(s0, s1, g1, dinv_rev, dinv_rev, b1p, w2p)
    s0, s1 = _edge_scatter(g2, eb3, ea3)
    g3 = kmid(s0, s1, g2, dinv_rev, dinv_fwd, b2p, wpp)
    s0, s1 = _edge_scatter(g3, ea3, eb3)
    g4 = kmid(s0, s1, g3, dinv_fwd, dinv_rev, bpp, w3p)
    s0, s1 = _edge_scatter(g4, eb3, ea3)
    g5, vhead, pnhead = pl.pallas_call(
        _k5_body,
        out_shape=(_node_rows,
                   jax.ShapeDtypeStruct((1, LANES), _f32),
                   jax.ShapeDtypeStruct((1, LANES), _f32)),
    )(s0, s1, g4, dinv_rev, b3p, wop, wvp, wdp, bvp, bdp)
    s0, s1 = _edge_scatter(g5, eb3, ea3)
    p5 = pl.pallas_call(_k6_body, out_shape=_node_rows)(
        s0, s1, g5, dinv_rev, bop)

    probs = jnp.concatenate([p5[:N, :1], pnhead[:1, :1]], axis=0)
    return (probs, vhead[0, :1])
